# baseline (speedup 1.0000x reference)
import functools

import jax
import jax.numpy as jnp
from jax import lax
from jax.experimental import pallas as pl
from jax.experimental.pallas import tpu as pltpu
from jax.experimental.pallas import tpu_sc as plsc

N_OBJ = 10000
N_EDGE = 320000
D = 128
H = 128

NC = 2
NS = 16
NW = NC * NS
CHUNK = 80
CW = 16

EPW = N_EDGE // NW
EPC = N_EDGE // NC
EPS = EPC // NS

N_OBJ_PAD = 10240
RPT = N_OBJ_PAD // NS


def _sc_mesh():
    return plsc.VectorSubcoreMesh(core_axis_name="c", subcore_axis_name="s")


def _sc_gather(obj, s_idx, o_idx):
    rows_t = jax.ShapeDtypeStruct((N_EDGE, D), jnp.float32)

    @functools.partial(
        pl.kernel,
        mesh=_sc_mesh(),
        out_type=[rows_t, rows_t],
        scratch_types=[
            pltpu.VMEM((CHUNK,), jnp.int32),
            pltpu.VMEM((CHUNK, D), jnp.float32),
            pltpu.SemaphoreType.DMA,
        ],
    )
    def gather_kernel(obj_hbm, s_hbm, o_hbm, s_out, o_out, idx_v, rows_v, sem):
        wid = lax.axis_index("s") * NC + lax.axis_index("c")
        base = wid * EPW

        @pl.loop(0, EPW, step=CHUNK)
        def _(off):
            b = base + off
            for i_hbm, r_out in ((s_hbm, s_out), (o_hbm, o_out)):
                pltpu.sync_copy(i_hbm.at[pl.ds(b, CHUNK)], idx_v)
                pltpu.async_copy(obj_hbm.at[idx_v], rows_v, sem).wait()
                pltpu.sync_copy(rows_v, r_out.at[pl.ds(b, CHUNK)])

    return gather_kernel(obj, s_idx, o_idx)


NBP = N_OBJ_PAD
EBP = 512
NEB = N_EDGE // EBP


def _pool_node_body(sidx_ref, oidx_ref, ns_ref, no_ref, w2a_ref, b2a_ref,
                    w2b_ref, b2b_ref, out_ref, acc, cnt):
    e = pl.program_id(0)

    @pl.when(e == 0)
    def _():
        acc[...] = jnp.zeros_like(acc)
        cnt[...] = jnp.zeros_like(cnt)

    node_ids = lax.broadcasted_iota(jnp.int32, (NBP, 1), 0)
    oh_s = (node_ids == sidx_ref[...][None, :]).astype(jnp.bfloat16)
    oh_o = (node_ids == oidx_ref[...][None, :]).astype(jnp.bfloat16)
    oh = jnp.concatenate([oh_s, oh_o], axis=1)
    msg = jnp.concatenate([ns_ref[...], no_ref[...]],
                          axis=0).astype(jnp.bfloat16)
    acc[...] += jnp.dot(oh, msg, preferred_element_type=jnp.float32)
    cnt[...] += jnp.dot(oh, jnp.ones((2 * EBP, D), jnp.bfloat16),
                        preferred_element_type=jnp.float32)

    @pl.when(e == NEB - 1)
    def _():
        pooled = acc[...] / jnp.maximum(cnt[:, 0:1], 1.0)
        h2 = jnp.maximum(jnp.dot(pooled, w2a_ref[...],
                                 preferred_element_type=jnp.float32)
                         + b2a_ref[...], 0.0)
        out_ref[...] = jnp.maximum(jnp.dot(h2, w2b_ref[...],
                                           preferred_element_type=jnp.float32)
                                   + b2b_ref[...], 0.0)


def _pool_node_mlp(s_idx2, o_idx2, new_s, new_o, W2a, b2a, W2b, b2b):
    idx_spec = pl.BlockSpec((EBP,), lambda e: (e,))
    msg_spec = pl.BlockSpec((EBP, D), lambda e: (e, 0))
    return pl.pallas_call(
        _pool_node_body,
        grid=(NEB,),
        in_specs=[
            idx_spec, idx_spec, msg_spec, msg_spec,
            pl.BlockSpec((H, H), lambda e: (0, 0)),
            pl.BlockSpec((1, H), lambda e: (0, 0)),
            pl.BlockSpec((H, D), lambda e: (0, 0)),
            pl.BlockSpec((1, D), lambda e: (0, 0)),
        ],
        out_specs=pl.BlockSpec((NBP, D), lambda e: (0, 0)),
        out_shape=jax.ShapeDtypeStruct((N_OBJ_PAD, D), jnp.float32),
        scratch_shapes=[
            pltpu.VMEM((NBP, D), jnp.float32),
            pltpu.VMEM((NBP, D), jnp.float32),
        ],
    )(s_idx2, o_idx2, new_s, new_o, W2a, b2a.reshape(1, H), W2b,
      b2b.reshape(1, D))


EB = 1280


def _edge_mlp_body(s_ref, p_ref, o_ref, ws_ref, wp_ref, wo_ref, b1a_ref,
                   w1b_ref, b1b_ref, ns_ref, np_ref, no_ref):
    acc = jnp.dot(s_ref[...], ws_ref[...], preferred_element_type=jnp.float32)
    acc = acc + jnp.dot(p_ref[...], wp_ref[...],
                        preferred_element_type=jnp.float32)
    acc = acc + jnp.dot(o_ref[...], wo_ref[...],
                        preferred_element_type=jnp.float32)
    h = jnp.maximum(acc + b1a_ref[...], 0.0)
    t = jnp.maximum(jnp.dot(h, w1b_ref[...],
                            preferred_element_type=jnp.float32)
                    + b1b_ref[...], 0.0)
    ns_ref[...] = t[:, :H]
    np_ref[...] = t[:, H:H + D]
    no_ref[...] = t[:, H + D:]


def _edge_mlp(s_rows, p_rows, o_rows, W1a, b1a, W1b, b1b):
    ws, wp, wo = W1a[:D], W1a[D:2 * D], W1a[2 * D:]
    row_spec = pl.BlockSpec((EB, D), lambda i: (i, 0))
    w_spec = pl.BlockSpec((D, H), lambda i: (0, 0))
    out_t = jax.ShapeDtypeStruct((N_EDGE, D), jnp.float32)
    return pl.pallas_call(
        _edge_mlp_body,
        grid=(N_EDGE // EB,),
        in_specs=[
            row_spec, row_spec, row_spec,
            w_spec, w_spec, w_spec,
            pl.BlockSpec((1, H), lambda i: (0, 0)),
            pl.BlockSpec((H, 2 * H + D), lambda i: (0, 0)),
            pl.BlockSpec((1, 2 * H + D), lambda i: (0, 0)),
        ],
        out_specs=[row_spec, row_spec, row_spec],
        out_shape=[out_t, out_t, out_t],
    )(s_rows, p_rows, o_rows, ws, wp, wo, b1a.reshape(1, H), W1b,
      b1b.reshape(1, 2 * H + D))


def _gcn_layer(obj, pred, s_idx, o_idx, s_idx2, o_idx2, W1a, b1a, W1b, b1b,
               W2a, b2a, W2b, b2b):
    s_rows, o_rows = _sc_gather(obj, s_idx, o_idx)
    new_s, new_p, new_o = _edge_mlp(s_rows, pred, o_rows, W1a, b1a, W1b, b1b)
    new_obj = _pool_node_mlp(s_idx2, o_idx2, new_s, new_o, W2a, b2a, W2b, b2b)
    return new_obj, new_p


def kernel(obj_vecs, pred_vecs, edges,
           l0_W1a, l0_b1a, l0_W1b, l0_b1b, l0_W2a, l0_b2a, l0_W2b, l0_b2b,
           l1_W1a, l1_b1a, l1_W1b, l1_b1b, l1_W2a, l1_b2a, l1_W2b, l1_b2b):
    s_idx = edges[:, 0]
    o_idx = edges[:, 1]
    s_idx2 = s_idx
    o_idx2 = o_idx

    obj1, pred1 = _gcn_layer(obj_vecs, pred_vecs, s_idx, o_idx, s_idx2,
                             o_idx2, l0_W1a, l0_b1a, l0_W1b, l0_b1b,
                             l0_W2a, l0_b2a, l0_W2b, l0_b2b)
    obj2, pred2 = _gcn_layer(obj1, pred1, s_idx, o_idx, s_idx2, o_idx2,
                             l1_W1a, l1_b1a, l1_W1b, l1_b1b,
                             l1_W2a, l1_b2a, l1_W2b, l1_b2b)
    return (obj2[:N_OBJ], pred2)


# speedup vs baseline: 1.3356x; 1.3356x over previous
import functools

import jax
import jax.numpy as jnp
from jax import lax
from jax.experimental import pallas as pl
from jax.experimental.pallas import tpu as pltpu
from jax.experimental.pallas import tpu_sc as plsc

N_OBJ = 10000
N_EDGE = 320000
D = 128
H = 128

NC = 2
NS = 16
NW = NC * NS
CHUNK = 80
CW = 16

EPW = N_EDGE // NW
EPC = N_EDGE // NC
EPS = EPC // NS

N_OBJ_PAD = 10240
RPT = N_OBJ_PAD // NS


def _sc_mesh():
    return plsc.VectorSubcoreMesh(core_axis_name="c", subcore_axis_name="s")


def _sc_gather(obj, s_idx, o_idx):
    rows_t = jax.ShapeDtypeStruct((N_EDGE, D), jnp.float32)

    @functools.partial(
        pl.kernel,
        mesh=_sc_mesh(),
        out_type=[rows_t, rows_t],
        scratch_types=[
            pltpu.VMEM((CHUNK,), jnp.int32),
            pltpu.VMEM((CHUNK, D), jnp.float32),
            pltpu.SemaphoreType.DMA,
        ],
    )
    def gather_kernel(obj_hbm, s_hbm, o_hbm, s_out, o_out, idx_v, rows_v, sem):
        wid = lax.axis_index("s") * NC + lax.axis_index("c")
        base = wid * EPW

        @pl.loop(0, EPW, step=CHUNK)
        def _(off):
            b = base + off
            for i_hbm, r_out in ((s_hbm, s_out), (o_hbm, o_out)):
                pltpu.sync_copy(i_hbm.at[pl.ds(b, CHUNK)], idx_v)
                pltpu.async_copy(obj_hbm.at[idx_v], rows_v, sem).wait()
                pltpu.sync_copy(rows_v, r_out.at[pl.ds(b, CHUNK)])

    return gather_kernel(obj, s_idx, o_idx)


NBP = N_OBJ_PAD
EBP = 512
NEB = N_EDGE // EBP


def _pool_node_body(sidx_ref, oidx_ref, ns_ref, no_ref, w2a_ref, b2a_ref,
                    w2b_ref, b2b_ref, out_ref, acc, cnt):
    e = pl.program_id(0)

    @pl.when(e == 0)
    def _():
        acc[...] = jnp.zeros_like(acc)
        cnt[...] = jnp.zeros_like(cnt)

    node_ids = lax.broadcasted_iota(jnp.int32, (NBP, 1), 0)
    oh_s = (node_ids == sidx_ref[...][None, :]).astype(jnp.bfloat16)
    oh_o = (node_ids == oidx_ref[...][None, :]).astype(jnp.bfloat16)
    acc[...] += (jnp.dot(oh_s, ns_ref[...].astype(jnp.bfloat16),
                         preferred_element_type=jnp.float32)
                 + jnp.dot(oh_o, no_ref[...].astype(jnp.bfloat16),
                           preferred_element_type=jnp.float32))
    csum = (jnp.sum(oh_s, axis=1, keepdims=True, dtype=jnp.float32)
            + jnp.sum(oh_o, axis=1, keepdims=True, dtype=jnp.float32))
    cnt[...] += jnp.broadcast_to(csum, (NBP, D))

    @pl.when(e == NEB - 1)
    def _():
        pooled = acc[...] / jnp.maximum(cnt[:, 0:1], 1.0)
        h2 = jnp.maximum(jnp.dot(pooled, w2a_ref[...],
                                 preferred_element_type=jnp.float32)
                         + b2a_ref[...], 0.0)
        out_ref[...] = jnp.maximum(jnp.dot(h2, w2b_ref[...],
                                           preferred_element_type=jnp.float32)
                                   + b2b_ref[...], 0.0)


def _pool_node_mlp(s_idx2, o_idx2, new_s, new_o, W2a, b2a, W2b, b2b):
    idx_spec = pl.BlockSpec((EBP,), lambda e: (e,))
    msg_spec = pl.BlockSpec((EBP, D), lambda e: (e, 0))
    return pl.pallas_call(
        _pool_node_body,
        grid=(NEB,),
        in_specs=[
            idx_spec, idx_spec, msg_spec, msg_spec,
            pl.BlockSpec((H, H), lambda e: (0, 0)),
            pl.BlockSpec((1, H), lambda e: (0, 0)),
            pl.BlockSpec((H, D), lambda e: (0, 0)),
            pl.BlockSpec((1, D), lambda e: (0, 0)),
        ],
        out_specs=pl.BlockSpec((NBP, D), lambda e: (0, 0)),
        out_shape=jax.ShapeDtypeStruct((N_OBJ_PAD, D), jnp.float32),
        scratch_shapes=[
            pltpu.VMEM((NBP, D), jnp.float32),
            pltpu.VMEM((NBP, D), jnp.float32),
        ],
    )(s_idx2, o_idx2, new_s, new_o, W2a, b2a.reshape(1, H), W2b,
      b2b.reshape(1, D))


EB = 1280


def _edge_mlp_body(s_ref, p_ref, o_ref, ws_ref, wp_ref, wo_ref, b1a_ref,
                   w1b_ref, b1b_ref, ns_ref, np_ref, no_ref):
    acc = jnp.dot(s_ref[...], ws_ref[...], preferred_element_type=jnp.float32)
    acc = acc + jnp.dot(p_ref[...], wp_ref[...],
                        preferred_element_type=jnp.float32)
    acc = acc + jnp.dot(o_ref[...], wo_ref[...],
                        preferred_element_type=jnp.float32)
    h = jnp.maximum(acc + b1a_ref[...], 0.0)
    t = jnp.maximum(jnp.dot(h, w1b_ref[...],
                            preferred_element_type=jnp.float32)
                    + b1b_ref[...], 0.0)
    ns_ref[...] = t[:, :H]
    np_ref[...] = t[:, H:H + D]
    no_ref[...] = t[:, H + D:]


def _edge_mlp(s_rows, p_rows, o_rows, W1a, b1a, W1b, b1b):
    ws, wp, wo = W1a[:D], W1a[D:2 * D], W1a[2 * D:]
    row_spec = pl.BlockSpec((EB, D), lambda i: (i, 0))
    w_spec = pl.BlockSpec((D, H), lambda i: (0, 0))
    out_t = jax.ShapeDtypeStruct((N_EDGE, D), jnp.float32)
    return pl.pallas_call(
        _edge_mlp_body,
        grid=(N_EDGE // EB,),
        in_specs=[
            row_spec, row_spec, row_spec,
            w_spec, w_spec, w_spec,
            pl.BlockSpec((1, H), lambda i: (0, 0)),
            pl.BlockSpec((H, 2 * H + D), lambda i: (0, 0)),
            pl.BlockSpec((1, 2 * H + D), lambda i: (0, 0)),
        ],
        out_specs=[row_spec, row_spec, row_spec],
        out_shape=[out_t, out_t, out_t],
    )(s_rows, p_rows, o_rows, ws, wp, wo, b1a.reshape(1, H), W1b,
      b1b.reshape(1, 2 * H + D))


def _gcn_layer(obj, pred, s_idx, o_idx, s_idx2, o_idx2, W1a, b1a, W1b, b1b,
               W2a, b2a, W2b, b2b):
    s_rows, o_rows = _sc_gather(obj, s_idx, o_idx)
    new_s, new_p, new_o = _edge_mlp(s_rows, pred, o_rows, W1a, b1a, W1b, b1b)
    new_obj = _pool_node_mlp(s_idx2, o_idx2, new_s, new_o, W2a, b2a, W2b, b2b)
    return new_obj, new_p


def kernel(obj_vecs, pred_vecs, edges,
           l0_W1a, l0_b1a, l0_W1b, l0_b1b, l0_W2a, l0_b2a, l0_W2b, l0_b2b,
           l1_W1a, l1_b1a, l1_W1b, l1_b1b, l1_W2a, l1_b2a, l1_W2b, l1_b2b):
    s_idx = edges[:, 0]
    o_idx = edges[:, 1]
    s_idx2 = s_idx
    o_idx2 = o_idx

    obj1, pred1 = _gcn_layer(obj_vecs, pred_vecs, s_idx, o_idx, s_idx2,
                             o_idx2, l0_W1a, l0_b1a, l0_W1b, l0_b1b,
                             l0_W2a, l0_b2a, l0_W2b, l0_b2b)
    obj2, pred2 = _gcn_layer(obj1, pred1, s_idx, o_idx, s_idx2, o_idx2,
                             l1_W1a, l1_b1a, l1_W1b, l1_b1b,
                             l1_W2a, l1_b2a, l1_W2b, l1_b2b)
    return (obj2[:N_OBJ], pred2)


# speedup vs baseline: 1.3359x; 1.0003x over previous
import functools

import jax
import jax.numpy as jnp
from jax import lax
from jax.experimental import pallas as pl
from jax.experimental.pallas import tpu as pltpu
from jax.experimental.pallas import tpu_sc as plsc

N_OBJ = 10000
N_EDGE = 320000
D = 128
H = 128

NC = 2
NS = 16
NW = NC * NS
CHUNK = 80

EPW = N_EDGE // NW
N_OBJ_PAD = 10240


def _sc_mesh():
    return plsc.VectorSubcoreMesh(core_axis_name="c", subcore_axis_name="s")


def _sc_gather(obj, s_idx, o_idx):
    rows_t = jax.ShapeDtypeStruct((N_EDGE, D), jnp.float32)

    @functools.partial(
        pl.kernel,
        mesh=_sc_mesh(),
        out_type=[rows_t, rows_t],
        scratch_types=[
            pltpu.VMEM((CHUNK,), jnp.int32),
            pltpu.VMEM((CHUNK, D), jnp.float32),
            pltpu.SemaphoreType.DMA,
        ],
    )
    def gather_kernel(obj_hbm, s_hbm, o_hbm, s_out, o_out, idx_v, rows_v, sem):
        wid = lax.axis_index("s") * NC + lax.axis_index("c")
        base = wid * EPW

        @pl.loop(0, EPW, step=CHUNK)
        def _(off):
            b = base + off
            for i_hbm, r_out in ((s_hbm, s_out), (o_hbm, o_out)):
                pltpu.sync_copy(i_hbm.at[pl.ds(b, CHUNK)], idx_v)
                pltpu.async_copy(obj_hbm.at[idx_v], rows_v, sem).wait()
                pltpu.sync_copy(rows_v, r_out.at[pl.ds(b, CHUNK)])

    return gather_kernel(obj, s_idx, o_idx)


NBP = N_OBJ_PAD
EBP = 512
NEB = N_EDGE // EBP


def _pool_node_body(sidx_ref, oidx_ref, ns_ref, no_ref, w2a_ref, b2a_ref,
                    w2b_ref, b2b_ref, out_ref, acc, cnt):
    e = pl.program_id(0)

    @pl.when(e == 0)
    def _():
        acc[...] = jnp.zeros_like(acc)
        cnt[...] = jnp.zeros_like(cnt)

    node_ids = lax.broadcasted_iota(jnp.int32, (NBP, 1), 0)
    oh_s = (node_ids == sidx_ref[...][None, :]).astype(jnp.bfloat16)
    oh_o = (node_ids == oidx_ref[...][None, :]).astype(jnp.bfloat16)
    acc[...] += (jnp.dot(oh_s, ns_ref[...].astype(jnp.bfloat16),
                         preferred_element_type=jnp.float32)
                 + jnp.dot(oh_o, no_ref[...].astype(jnp.bfloat16),
                           preferred_element_type=jnp.float32))
    csum = (jnp.sum(oh_s, axis=1, keepdims=True, dtype=jnp.float32)
            + jnp.sum(oh_o, axis=1, keepdims=True, dtype=jnp.float32))
    cnt[...] += jnp.broadcast_to(csum, (NBP, D))

    @pl.when(e == NEB - 1)
    def _():
        pooled = acc[...] / jnp.maximum(cnt[:, 0:1], 1.0)
        h2 = jnp.maximum(jnp.dot(pooled, w2a_ref[...],
                                 preferred_element_type=jnp.float32)
                         + b2a_ref[...], 0.0)
        out_ref[...] = jnp.maximum(jnp.dot(h2, w2b_ref[...],
                                           preferred_element_type=jnp.float32)
                                   + b2b_ref[...], 0.0)


def _pool_node_mlp(s_idx2, o_idx2, new_s, new_o, W2a, b2a, W2b, b2b):
    idx_spec = pl.BlockSpec((EBP,), lambda e: (e,))
    msg_spec = pl.BlockSpec((EBP, D), lambda e: (e, 0))
    return pl.pallas_call(
        _pool_node_body,
        grid=(NEB,),
        in_specs=[
            idx_spec, idx_spec, msg_spec, msg_spec,
            pl.BlockSpec((H, H), lambda e: (0, 0)),
            pl.BlockSpec((1, H), lambda e: (0, 0)),
            pl.BlockSpec((H, D), lambda e: (0, 0)),
            pl.BlockSpec((1, D), lambda e: (0, 0)),
        ],
        out_specs=pl.BlockSpec((NBP, D), lambda e: (0, 0)),
        out_shape=jax.ShapeDtypeStruct((N_OBJ_PAD, D), jnp.float32),
        scratch_shapes=[
            pltpu.VMEM((NBP, D), jnp.float32),
            pltpu.VMEM((NBP, D), jnp.float32),
        ],
    )(s_idx2, o_idx2, new_s, new_o, W2a, b2a.reshape(1, H), W2b,
      b2b.reshape(1, D))


EB = 1280


def _edge_mlp_body(s_ref, p_ref, o_ref, ws_ref, wp_ref, wo_ref, b1a_ref,
                   w1b_ref, b1b_ref, ns_ref, np_ref, no_ref):
    acc = jnp.dot(s_ref[...], ws_ref[...], preferred_element_type=jnp.float32)
    acc = acc + jnp.dot(p_ref[...], wp_ref[...],
                        preferred_element_type=jnp.float32)
    acc = acc + jnp.dot(o_ref[...], wo_ref[...],
                        preferred_element_type=jnp.float32)
    h = jnp.maximum(acc + b1a_ref[...], 0.0)
    t = jnp.maximum(jnp.dot(h, w1b_ref[...],
                            preferred_element_type=jnp.float32)
                    + b1b_ref[...], 0.0)
    ns_ref[...] = t[:, :H]
    np_ref[...] = t[:, H:H + D]
    no_ref[...] = t[:, H + D:]


def _edge_mlp(s_rows, p_rows, o_rows, W1a, b1a, W1b, b1b):
    ws, wp, wo = W1a[:D], W1a[D:2 * D], W1a[2 * D:]
    row_spec = pl.BlockSpec((EB, D), lambda i: (i, 0))
    w_spec = pl.BlockSpec((D, H), lambda i: (0, 0))
    out_t = jax.ShapeDtypeStruct((N_EDGE, D), jnp.float32)
    return pl.pallas_call(
        _edge_mlp_body,
        grid=(N_EDGE // EB,),
        in_specs=[
            row_spec, row_spec, row_spec,
            w_spec, w_spec, w_spec,
            pl.BlockSpec((1, H), lambda i: (0, 0)),
            pl.BlockSpec((H, 2 * H + D), lambda i: (0, 0)),
            pl.BlockSpec((1, 2 * H + D), lambda i: (0, 0)),
        ],
        out_specs=[row_spec, row_spec, row_spec],
        out_shape=[out_t, out_t, out_t],
    )(s_rows, p_rows, o_rows, ws, wp, wo, b1a.reshape(1, H), W1b,
      b1b.reshape(1, 2 * H + D))


def _gcn_layer(obj, pred, s_idx, o_idx, W1a, b1a, W1b, b1b,
               W2a, b2a, W2b, b2b):
    s_rows, o_rows = _sc_gather(obj, s_idx, o_idx)
    new_s, new_p, new_o = _edge_mlp(s_rows, pred, o_rows, W1a, b1a, W1b, b1b)
    new_obj = _pool_node_mlp(s_idx, o_idx, new_s, new_o, W2a, b2a, W2b, b2b)
    return new_obj, new_p


def kernel(obj_vecs, pred_vecs, edges,
           l0_W1a, l0_b1a, l0_W1b, l0_b1b, l0_W2a, l0_b2a, l0_W2b, l0_b2b,
           l1_W1a, l1_b1a, l1_W1b, l1_b1b, l1_W2a, l1_b2a, l1_W2b, l1_b2b):
    s_idx = edges[:, 0]
    o_idx = edges[:, 1]

    obj1, pred1 = _gcn_layer(obj_vecs, pred_vecs, s_idx, o_idx,
                             l0_W1a, l0_b1a, l0_W1b, l0_b1b,
                             l0_W2a, l0_b2a, l0_W2b, l0_b2b)
    obj2, pred2 = _gcn_layer(obj1, pred1, s_idx, o_idx,
                             l1_W1a, l1_b1a, l1_W1b, l1_b1b,
                             l1_W2a, l1_b2a, l1_W2b, l1_b2b)
    return (obj2[:N_OBJ], pred2)
